# baseline (speedup 1.0000x reference)
import jax
import jax.numpy as jnp
from jax.experimental import pallas as pl

_N = 65536
_B = 256
_NP = _N // _B
_D_IN = 16
_H = 64
_K = 16
_OUT = 8


def _elu(x):
    return jnp.where(x > 0, x, jnp.exp(x) - 1.0)


def _bf(x):
    return x.astype(jnp.bfloat16)


def _mm(a, b):
    return jax.lax.dot_general(_bf(a), _bf(b), (((1,), (0,)), ((), ())),
                               preferred_element_type=jnp.float32)


def _split(a):
    ah = _bf(a)
    al = _bf(a - ah.astype(jnp.float32))
    return ah, al


def _mm3(a, b):
    ah, al = _split(a)
    bh, bl = _split(b)
    d = lambda x, y: jax.lax.dot_general(x, y, (((1,), (0,)), ((), ())),
                                         preferred_element_type=jnp.float32)
    return d(ah, bh) + d(ah, bl) + d(al, bh)


def _gram3(a):
    ah, al = _split(a)
    d = lambda x, y: jax.lax.dot_general(x, y, (((1,), (1,)), ((), ())),
                                         preferred_element_type=jnp.float32)
    cross = d(ah, al)
    return d(ah, ah) + cross + cross.T


def _edgeconv(hg, wd, wq, ba, wb, bb):
    hb = _bf(hg)
    gram = jax.lax.dot_general(hb, hb, (((1,), (1,)), ((), ())),
                               preferred_element_type=jnp.float32)
    sq = hg * hg
    d2r = jnp.sum(sq, axis=1, keepdims=True)
    sqh = _bf(sq)
    sql = _bf(sq - sqh.astype(jnp.float32))
    ones = jnp.ones((1, _H), jnp.bfloat16)
    d2c = (jax.lax.dot_general(ones, sqh, (((1,), (1,)), ((), ())),
                               preferred_element_type=jnp.float32) +
           jax.lax.dot_general(ones, sql, (((1,), (1,)), ((), ())),
                               preferred_element_type=jnp.float32))
    dist = (d2r + d2c) - 2.0 * gram
    ri = jax.lax.broadcasted_iota(jnp.int32, (_NP, _NP), 0)
    ci = jax.lax.broadcasted_iota(jnp.int32, (_NP, _NP), 1)
    dist = jnp.where(ri == ci, dist + 1e9, dist)

    pre_i = _mm(hg, wd) + ba
    q = _mm(hg, wq)
    qb = _bf(q)

    acc = jnp.zeros((_NP, _H), jnp.float32)
    d = dist
    for _ in range(_K):
        m = jnp.min(d, axis=1, keepdims=True)
        j = jnp.min(jnp.where(d == m, ci, _NP), axis=1, keepdims=True)
        oh = ci == j
        d = jnp.where(oh, d + 1e9, d)
        sel = oh.astype(jnp.bfloat16)
        qg = jax.lax.dot_general(sel, qb, (((1,), (0,)), ((), ())),
                                 preferred_element_type=jnp.float32)
        t = _elu(pre_i + qg)
        acc = acc + _elu(_mm(t, wb) + bb)
    return acc


def _net_body(x_ref,
              wi0, bi0, wi1, bi1, wi2, bi2,
              wd1, wq1, ba1, wb1, bb1,
              wd2, wq2, ba2, wb2, bb2,
              wo0, bo0, wo1, bo1, wo2, bo2,
              out_ref):
    xg = x_ref[0]
    h = _elu(_mm(xg, wi0[...]) + bi0[...])
    h = _elu(_mm(h, wi1[...]) + bi1[...])
    h = _elu(_mm(h, wi2[...]) + bi2[...])
    h = _edgeconv(h, wd1[...], wq1[...], ba1[...], wb1[...], bb1[...])
    h = _edgeconv(h, wd2[...], wq2[...], ba2[...], wb2[...], bb2[...])
    p = jnp.max(h, axis=0, keepdims=True)
    l = _elu(_mm(p, wo0[...]) + bo0[...])
    l = _elu(_mm(l, wo1[...]) + bo1[...])
    l = _mm(l, wo2[...]) + bo2[...]
    mx = jnp.max(l, axis=1, keepdims=True)
    lse = jnp.log(jnp.sum(jnp.exp(l - mx), axis=1, keepdims=True)) + mx
    out_ref[0] = l - lse


def kernel(x, batch, params):
    del batch

    (wi0, bi0), (wi1, bi1), (wi2, bi2) = params['in']
    (wa1, ba1), (wb1, bb1) = params['ec1']
    (wa2, ba2), (wb2, bb2) = params['ec2']
    (wo0, bo0), (wo1, bo1), (wo2, bo2) = params['out']

    wd1 = wa1[:_H] - wa1[_H:]
    wq1 = wa1[_H:]
    wd2 = wa2[:_H] - wa2[_H:]
    wq2 = wa2[_H:]

    ws = [wi0, bi0.reshape(1, -1), wi1, bi1.reshape(1, -1),
          wi2, bi2.reshape(1, -1),
          wd1, wq1, ba1.reshape(1, -1), wb1, bb1.reshape(1, -1),
          wd2, wq2, ba2.reshape(1, -1), wb2, bb2.reshape(1, -1),
          wo0, bo0.reshape(1, -1), wo1, bo1.reshape(1, -1),
          wo2, bo2.reshape(1, -1)]

    def _const_spec(w):
        nd = w.ndim
        return pl.BlockSpec(w.shape, lambda i, _nd=nd: (0,) * _nd)

    out = pl.pallas_call(
        _net_body,
        grid=(_B,),
        in_specs=[pl.BlockSpec((1, _NP, _D_IN), lambda i: (i, 0, 0))] +
                 [_const_spec(w) for w in ws],
        out_specs=pl.BlockSpec((1, 1, _OUT), lambda i: (i, 0, 0)),
        out_shape=jax.ShapeDtypeStruct((_B, 1, _OUT), jnp.float32),
    )(x.reshape(_B, _NP, _D_IN), *ws)
    return out.reshape(_B, _OUT)


# speedup vs baseline: 1.2747x; 1.2747x over previous
import jax
import jax.numpy as jnp
from jax.experimental import pallas as pl

_N = 65536
_B = 256
_NP = _N // _B
_D_IN = 16
_H = 64
_K = 16
_OUT = 8
_GPB = 2


def _elu(x):
    return jnp.where(x > 0, x, jnp.exp(x) - 1.0)


def _bf(x):
    return x.astype(jnp.bfloat16)


def _mm(a, b):
    return jax.lax.dot_general(_bf(a), _bf(b), (((1,), (0,)), ((), ())),
                               preferred_element_type=jnp.float32)


def _split(a):
    ah = _bf(a)
    al = _bf(a - ah.astype(jnp.float32))
    return ah, al


def _mm3(a, b):
    ah, al = _split(a)
    bh, bl = _split(b)
    d = lambda x, y: jax.lax.dot_general(x, y, (((1,), (0,)), ((), ())),
                                         preferred_element_type=jnp.float32)
    return d(ah, bh) + d(ah, bl) + d(al, bh)


def _gram3(a):
    ah, al = _split(a)
    d = lambda x, y: jax.lax.dot_general(x, y, (((1,), (1,)), ((), ())),
                                         preferred_element_type=jnp.float32)
    cross = d(ah, al)
    return d(ah, ah) + cross + cross.T


def _edgeconv(hg, wd, wq, ba, wb, bb):
    hb = _bf(hg)
    gram = jax.lax.dot_general(hb, hb, (((1,), (1,)), ((), ())),
                               preferred_element_type=jnp.float32)
    sq = hg * hg
    d2r = jnp.sum(sq, axis=1, keepdims=True)
    sqh = _bf(sq)
    sql = _bf(sq - sqh.astype(jnp.float32))
    ones = jnp.ones((1, _H), jnp.bfloat16)
    d2c = (jax.lax.dot_general(ones, sqh, (((1,), (1,)), ((), ())),
                               preferred_element_type=jnp.float32) +
           jax.lax.dot_general(ones, sql, (((1,), (1,)), ((), ())),
                               preferred_element_type=jnp.float32))
    dist = (d2r + d2c) - 2.0 * gram
    ri = jax.lax.broadcasted_iota(jnp.int32, (_NP, _NP), 0).astype(jnp.float32)
    ci = jax.lax.broadcasted_iota(jnp.int32, (_NP, _NP), 1).astype(jnp.float32)
    dist = jnp.where(ri == ci, dist + 1e9, dist)

    pre_i = _mm(hg, wd) + ba
    q = _mm(hg, wq)
    qb = _bf(q)

    acc = jnp.zeros((_NP, _H), jnp.float32)
    d = dist
    for r in range(_K):
        m = jnp.min(d, axis=1, keepdims=True)
        j = jnp.min(jnp.where(d == m, ci, float(_NP)), axis=1, keepdims=True)
        oh = ci == j
        if r + 1 < _K:
            d = jnp.where(oh, d + 1e9, d)
        sel = oh.astype(jnp.bfloat16)
        qg = jax.lax.dot_general(sel, qb, (((1,), (0,)), ((), ())),
                                 preferred_element_type=jnp.float32)
        t = _elu(pre_i + qg)
        acc = acc + _elu(_mm(t, wb) + bb)
    return acc


def _net_body(x_ref,
              wi0, bi0, wi1, bi1, wi2, bi2,
              wd1, wq1, ba1, wb1, bb1,
              wd2, wq2, ba2, wb2, bb2,
              wo0, bo0, wo1, bo1, wo2, bo2,
              out_ref):
    for g in range(_GPB):
        _one_graph(g, x_ref,
                   wi0, bi0, wi1, bi1, wi2, bi2,
                   wd1, wq1, ba1, wb1, bb1,
                   wd2, wq2, ba2, wb2, bb2,
                   wo0, bo0, wo1, bo1, wo2, bo2,
                   out_ref)


def _one_graph(g, x_ref,
               wi0, bi0, wi1, bi1, wi2, bi2,
               wd1, wq1, ba1, wb1, bb1,
               wd2, wq2, ba2, wb2, bb2,
               wo0, bo0, wo1, bo1, wo2, bo2,
               out_ref):
    xg = x_ref[g]
    h = _elu(_mm(xg, wi0[...]) + bi0[...])
    h = _elu(_mm(h, wi1[...]) + bi1[...])
    h = _elu(_mm(h, wi2[...]) + bi2[...])
    h = _edgeconv(h, wd1[...], wq1[...], ba1[...], wb1[...], bb1[...])
    h = _edgeconv(h, wd2[...], wq2[...], ba2[...], wb2[...], bb2[...])
    p = jnp.max(h, axis=0, keepdims=True)
    l = _elu(_mm(p, wo0[...]) + bo0[...])
    l = _elu(_mm(l, wo1[...]) + bo1[...])
    l = _mm(l, wo2[...]) + bo2[...]
    mx = jnp.max(l, axis=1, keepdims=True)
    lse = jnp.log(jnp.sum(jnp.exp(l - mx), axis=1, keepdims=True)) + mx
    out_ref[g] = l - lse


def kernel(x, batch, params):
    del batch

    (wi0, bi0), (wi1, bi1), (wi2, bi2) = params['in']
    (wa1, ba1), (wb1, bb1) = params['ec1']
    (wa2, ba2), (wb2, bb2) = params['ec2']
    (wo0, bo0), (wo1, bo1), (wo2, bo2) = params['out']

    wd1 = wa1[:_H] - wa1[_H:]
    wq1 = wa1[_H:]
    wd2 = wa2[:_H] - wa2[_H:]
    wq2 = wa2[_H:]

    ws = [wi0, bi0.reshape(1, -1), wi1, bi1.reshape(1, -1),
          wi2, bi2.reshape(1, -1),
          wd1, wq1, ba1.reshape(1, -1), wb1, bb1.reshape(1, -1),
          wd2, wq2, ba2.reshape(1, -1), wb2, bb2.reshape(1, -1),
          wo0, bo0.reshape(1, -1), wo1, bo1.reshape(1, -1),
          wo2, bo2.reshape(1, -1)]

    def _const_spec(w):
        nd = w.ndim
        return pl.BlockSpec(w.shape, lambda i, _nd=nd: (0,) * _nd)

    out = pl.pallas_call(
        _net_body,
        grid=(_B // _GPB,),
        in_specs=[pl.BlockSpec((_GPB, _NP, _D_IN), lambda i: (i, 0, 0))] +
                 [_const_spec(w) for w in ws],
        out_specs=pl.BlockSpec((_GPB, 1, _OUT), lambda i: (i, 0, 0)),
        out_shape=jax.ShapeDtypeStruct((_B, 1, _OUT), jnp.float32),
    )(x.reshape(_B, _NP, _D_IN), *ws)
    return out.reshape(_B, _OUT)
